# baseline (speedup 1.0000x reference)
import functools

import jax
import jax.numpy as jnp
from jax import lax
from jax.experimental import pallas as pl
from jax.experimental.pallas import tpu as pltpu
from jax.experimental.pallas import tpu_sc as plsc

N = 10000
D = 128
G = 64
DU = 16
BCH = 80
NCORES = 2
NSUB = 16
NW = NCORES * NSUB
WCH = 2
NS = NW * WCH * BCH
NT = N - NS
GROWS = G // NSUB
IDS_R = 80
IDS_C = 125


@functools.partial(
    pl.kernel,
    mesh=plsc.VectorSubcoreMesh(core_axis_name="c", subcore_axis_name="s"),
    out_type=jax.ShapeDtypeStruct((NCORES, G, D), jnp.float32),
    scratch_types=[
        pltpu.VMEM((WCH * BCH, D), jnp.float32),
        pltpu.VMEM((WCH * BCH,), jnp.int32),
        pltpu.VMEM((GROWS, D), jnp.float32),
        pltpu.VMEM_SHARED((G, D), jnp.float32),
        pltpu.SemaphoreType.DMA,
        pltpu.SemaphoreType.DMA,
        pltpu.SemaphoreType.DMA,
        pltpu.SemaphoreType.DMA,
    ],
)
def _sc_segment_sums(x_hbm, b1d_hbm, sums_out, xbuf, idxbuf, zsum, acc,
                     sem_i, sem_a, sem_b, sem_s):
    cid = lax.axis_index("c")
    sid = lax.axis_index("s")
    wid = cid * NSUB + sid

    zero16 = jnp.zeros((16,), jnp.float32)
    for r in range(GROWS):
        for c in range(D // 16):
            zsum[r, pl.ds(16 * c, 16)] = zero16

    base = WCH * wid
    xsem = [sem_a, sem_b]
    pltpu.async_copy(b1d_hbm.at[pl.ds(base * BCH, WCH * BCH)], idxbuf, sem_i)
    for j in range(WCH):
        pltpu.async_copy(
            x_hbm.at[pl.ds((base + j) * BCH, BCH)],
            xbuf.at[pl.ds(j * BCH, BCH)], xsem[j % 2])
    pltpu.sync_copy(zsum, acc.at[pl.ds(GROWS * sid, GROWS)])
    pltpu.make_async_copy(
        b1d_hbm.at[pl.ds(base * BCH, WCH * BCH)], idxbuf, sem_i).wait()
    plsc.subcore_barrier()

    for j in range(WCH):
        pltpu.make_async_copy(
            x_hbm.at[pl.ds((base + j) * BCH, BCH)],
            xbuf.at[pl.ds(j * BCH, BCH)], xsem[j % 2]).wait()
        pltpu.async_copy(
            xbuf.at[pl.ds(j * BCH, BCH)],
            acc.at[idxbuf.at[pl.ds(j * BCH, BCH)]], sem_s, add=True)
    for j in range(WCH):
        pltpu.make_async_copy(
            xbuf.at[pl.ds(j * BCH, BCH)],
            acc.at[idxbuf.at[pl.ds(j * BCH, BCH)]], sem_s).wait()

    plsc.subcore_barrier()

    @pl.when(sid == 0)
    def _():
        pltpu.sync_copy(acc, sums_out.at[cid])


def _tc_segsum_body(ids_ref, x_ref, o_r):
    ids = ids_ref[...]
    gids = jax.lax.broadcasted_iota(jnp.int32, (NT, G), 1)
    mask = (ids == gids).astype(jnp.float32)
    o_r[...] = jax.lax.dot_general(
        mask, x_ref[...], (((0,), (0,)), ((), ())),
        precision=jax.lax.Precision.HIGHEST,
        preferred_element_type=jnp.float32)


def _count_body(ids_ref, inv_r):
    ids = ids_ref[...]
    cntmat = jnp.concatenate(
        [jnp.sum((ids == g).astype(jnp.float32), axis=0, keepdims=True)
         for g in range(G)], axis=0)
    cnt = jnp.sum(cntmat, axis=1, keepdims=True)
    inv_r[...] = 1.0 / jnp.maximum(cnt, 1.0)


def _mlp_body(ps, ptc, inv_r, u_r, w1u, w1x, b1_r, w2, b2_r, o_r):
    mean = (ps[0] + ps[1] + ptc[...]) * inv_r[...]
    h = (jnp.dot(u_r[...], w1u[...], preferred_element_type=jnp.float32)
         + jnp.dot(mean, w1x[...], preferred_element_type=jnp.float32)
         + b1_r[...])
    h = jnp.where(h > 0, h, jnp.exp(jnp.minimum(h, 0.0)) - 1.0)
    o_r[...] = jnp.dot(h, w2[...], preferred_element_type=jnp.float32) + b2_r[...]


def kernel(x, edge_index, edge_attr, u, batch, W1, b1, W2, b2):
    del edge_index, edge_attr
    psums = _sc_segment_sums(x, batch)
    ids2d = jnp.pad(batch.reshape(IDS_R, IDS_C), ((0, 0), (0, 128 - IDS_C)),
                    constant_values=G)
    inv_cnt = pl.pallas_call(
        _count_body,
        out_shape=jax.ShapeDtypeStruct((G, 1), jnp.float32),
    )(ids2d)
    ptc = pl.pallas_call(
        _tc_segsum_body,
        out_shape=jax.ShapeDtypeStruct((G, D), jnp.float32),
    )(batch[NS:].reshape(NT, 1), x[NS:])
    out = pl.pallas_call(
        _mlp_body,
        out_shape=jax.ShapeDtypeStruct((G, W2.shape[1]), jnp.float32),
    )(psums, ptc, inv_cnt, u, W1[:DU], W1[DU:], b1.reshape(1, -1), W2,
      b2.reshape(1, -1))
    return out


# speedup vs baseline: 1.2367x; 1.2367x over previous
import functools

import jax
import jax.numpy as jnp
from jax import lax
from jax.experimental import pallas as pl
from jax.experimental.pallas import tpu as pltpu
from jax.experimental.pallas import tpu_sc as plsc

N = 10000
D = 128
G = 64
DU = 16
BCH = 80
NCORES = 2
NSUB = 16
NW = NCORES * NSUB
WCH = 3
NS = NW * WCH * BCH
NT = N - NS
GROWS = G // NSUB
IDS_R = 80
IDS_C = 125


@functools.partial(
    pl.kernel,
    mesh=plsc.VectorSubcoreMesh(core_axis_name="c", subcore_axis_name="s"),
    out_type=jax.ShapeDtypeStruct((NCORES, G, D), jnp.float32),
    scratch_types=[
        pltpu.VMEM((WCH * BCH, D), jnp.float32),
        pltpu.VMEM((WCH * BCH,), jnp.int32),
        pltpu.VMEM((GROWS, D), jnp.float32),
        pltpu.VMEM_SHARED((G, D), jnp.float32),
        pltpu.SemaphoreType.DMA,
        pltpu.SemaphoreType.DMA,
        pltpu.SemaphoreType.DMA,
        pltpu.SemaphoreType.DMA,
    ],
)
def _sc_segment_sums(x_hbm, b1d_hbm, sums_out, xbuf, idxbuf, zsum, acc,
                     sem_i, sem_a, sem_b, sem_s):
    cid = lax.axis_index("c")
    sid = lax.axis_index("s")
    wid = cid * NSUB + sid

    zero16 = jnp.zeros((16,), jnp.float32)
    for r in range(GROWS):
        for c in range(D // 16):
            zsum[r, pl.ds(16 * c, 16)] = zero16

    base = WCH * wid
    xsem = [sem_a, sem_b]
    pltpu.async_copy(b1d_hbm.at[pl.ds(base * BCH, WCH * BCH)], idxbuf, sem_i)
    for j in range(2):
        pltpu.async_copy(
            x_hbm.at[pl.ds((base + j) * BCH, BCH)],
            xbuf.at[pl.ds(j * BCH, BCH)], xsem[j % 2])
    pltpu.sync_copy(zsum, acc.at[pl.ds(GROWS * sid, GROWS)])
    pltpu.make_async_copy(
        b1d_hbm.at[pl.ds(base * BCH, WCH * BCH)], idxbuf, sem_i).wait()
    plsc.subcore_barrier()

    for j in range(WCH):
        pltpu.make_async_copy(
            x_hbm.at[pl.ds((base + j) * BCH, BCH)],
            xbuf.at[pl.ds(j * BCH, BCH)], xsem[j % 2]).wait()
        if j + 2 < WCH:
            pltpu.async_copy(
                x_hbm.at[pl.ds((base + j + 2) * BCH, BCH)],
                xbuf.at[pl.ds((j + 2) * BCH, BCH)], xsem[j % 2])
        pltpu.async_copy(
            xbuf.at[pl.ds(j * BCH, BCH)],
            acc.at[idxbuf.at[pl.ds(j * BCH, BCH)]], sem_s, add=True)
    for j in range(WCH):
        pltpu.make_async_copy(
            xbuf.at[pl.ds(j * BCH, BCH)],
            acc.at[idxbuf.at[pl.ds(j * BCH, BCH)]], sem_s).wait()

    plsc.subcore_barrier()

    @pl.when(sid == 0)
    def _():
        pltpu.sync_copy(acc, sums_out.at[cid])


def _tc_segsum_body(ids_ref, x_ref, o_r):
    ids = ids_ref[...]
    gids = jax.lax.broadcasted_iota(jnp.int32, (G, NT), 0)
    mask = (ids == gids).astype(jnp.float32)
    o_r[...] = jax.lax.dot_general(
        mask, x_ref[...], (((1,), (0,)), ((), ())),
        precision=jax.lax.Precision.HIGHEST,
        preferred_element_type=jnp.float32)


def _count_body(ids_ref, inv_r):
    ids = ids_ref[...]
    cntmat = jnp.concatenate(
        [jnp.sum((ids == g).astype(jnp.float32), axis=0, keepdims=True)
         for g in range(G)], axis=0)
    cnt = jnp.sum(cntmat, axis=1, keepdims=True)
    inv_r[...] = 1.0 / jnp.maximum(cnt, 1.0)


def _mlp_body(ps, ptc, inv_r, u_r, w1u, w1x, b1_r, w2, b2_r, o_r):
    mean = (ps[0] + ps[1] + ptc[...]) * inv_r[...]
    h = (jnp.dot(u_r[...], w1u[...], preferred_element_type=jnp.float32)
         + jnp.dot(mean, w1x[...], preferred_element_type=jnp.float32)
         + b1_r[...])
    h = jnp.where(h > 0, h, jnp.exp(jnp.minimum(h, 0.0)) - 1.0)
    o_r[...] = jnp.dot(h, w2[...], preferred_element_type=jnp.float32) + b2_r[...]


def kernel(x, edge_index, edge_attr, u, batch, W1, b1, W2, b2):
    del edge_index, edge_attr
    psums = _sc_segment_sums(x, batch)
    ids2d = jnp.pad(batch.reshape(IDS_R, IDS_C), ((0, 0), (0, 128 - IDS_C)),
                    constant_values=G)
    inv_cnt = pl.pallas_call(
        _count_body,
        out_shape=jax.ShapeDtypeStruct((G, 1), jnp.float32),
    )(ids2d)
    ptc = pl.pallas_call(
        _tc_segsum_body,
        out_shape=jax.ShapeDtypeStruct((G, D), jnp.float32),
    )(batch[NS:].reshape(1, NT), x[NS:])
    out = pl.pallas_call(
        _mlp_body,
        out_shape=jax.ShapeDtypeStruct((G, W2.shape[1]), jnp.float32),
    )(psums, ptc, inv_cnt, u, W1[:DU], W1[DU:], b1.reshape(1, -1), W2,
      b2.reshape(1, -1))
    return out


# speedup vs baseline: 1.3508x; 1.0922x over previous
import functools

import jax
import jax.numpy as jnp
from jax import lax
from jax.experimental import pallas as pl
from jax.experimental.pallas import tpu as pltpu
from jax.experimental.pallas import tpu_sc as plsc

N = 10000
D = 128
G = 64
DU = 16
BCH = 80
NCH = N // BCH
NCORES = 2
NSUB = 16
NW = NCORES * NSUB
WCH = 4
GROWS = G // NSUB
IDS_R = 80
IDS_C = 125


@functools.partial(
    pl.kernel,
    mesh=plsc.VectorSubcoreMesh(core_axis_name="c", subcore_axis_name="s"),
    out_type=jax.ShapeDtypeStruct((NCORES, G, D), jnp.float32),
    scratch_types=[
        pltpu.VMEM((WCH * BCH, D), jnp.float32),
        pltpu.VMEM((WCH * BCH,), jnp.int32),
        pltpu.VMEM((GROWS, D), jnp.float32),
        pltpu.VMEM_SHARED((G, D), jnp.float32),
        pltpu.SemaphoreType.DMA,
        pltpu.SemaphoreType.DMA,
        pltpu.SemaphoreType.DMA,
        pltpu.SemaphoreType.DMA,
    ],
)
def _sc_segment_sums(x_hbm, b1d_hbm, sums_out, xbuf, idxbuf, zsum, acc,
                     sem_i, sem_a, sem_b, sem_s):
    cid = lax.axis_index("c")
    sid = lax.axis_index("s")
    wid = cid * NSUB + sid

    zero16 = jnp.zeros((16,), jnp.float32)
    for r in range(GROWS):
        for c in range(D // 16):
            zsum[r, pl.ds(16 * c, 16)] = zero16

    base = jnp.minimum(WCH * wid, NCH - WCH)
    jmin = WCH * wid - base
    xsem = [sem_a, sem_b]
    pltpu.async_copy(b1d_hbm.at[pl.ds(base * BCH, WCH * BCH)], idxbuf, sem_i)
    for j in range(2):
        pltpu.async_copy(
            x_hbm.at[pl.ds((base + j) * BCH, BCH)],
            xbuf.at[pl.ds(j * BCH, BCH)], xsem[j % 2])
    pltpu.sync_copy(zsum, acc.at[pl.ds(GROWS * sid, GROWS)])
    pltpu.make_async_copy(
        b1d_hbm.at[pl.ds(base * BCH, WCH * BCH)], idxbuf, sem_i).wait()
    plsc.subcore_barrier()

    for j in range(WCH):
        pltpu.make_async_copy(
            x_hbm.at[pl.ds((base + j) * BCH, BCH)],
            xbuf.at[pl.ds(j * BCH, BCH)], xsem[j % 2]).wait()
        if j + 2 < WCH:
            pltpu.async_copy(
                x_hbm.at[pl.ds((base + j + 2) * BCH, BCH)],
                xbuf.at[pl.ds((j + 2) * BCH, BCH)], xsem[j % 2])

        @pl.when(j >= jmin)
        def _():
            pltpu.async_copy(
                xbuf.at[pl.ds(j * BCH, BCH)],
                acc.at[idxbuf.at[pl.ds(j * BCH, BCH)]], sem_s, add=True)
    for j in range(WCH):
        @pl.when(j >= jmin)
        def _():
            pltpu.make_async_copy(
                xbuf.at[pl.ds(j * BCH, BCH)],
                acc.at[idxbuf.at[pl.ds(j * BCH, BCH)]], sem_s).wait()

    plsc.subcore_barrier()

    @pl.when(sid == 0)
    def _():
        pltpu.sync_copy(acc, sums_out.at[cid])


def _count_body(ids_ref, inv_r):
    ids = ids_ref[...]
    cntmat = jnp.concatenate(
        [jnp.sum((ids == g).astype(jnp.float32), axis=0, keepdims=True)
         for g in range(G)], axis=0)
    cnt = jnp.sum(cntmat, axis=1, keepdims=True)
    inv_r[...] = 1.0 / jnp.maximum(cnt, 1.0)


def _mlp_body(ps, inv_r, u_r, w1u, w1x, b1_r, w2, b2_r, o_r):
    mean = (ps[0] + ps[1]) * inv_r[...]
    h = (jnp.dot(u_r[...], w1u[...], preferred_element_type=jnp.float32)
         + jnp.dot(mean, w1x[...], preferred_element_type=jnp.float32)
         + b1_r[...])
    h = jnp.where(h > 0, h, jnp.exp(jnp.minimum(h, 0.0)) - 1.0)
    o_r[...] = jnp.dot(h, w2[...], preferred_element_type=jnp.float32) + b2_r[...]


def kernel(x, edge_index, edge_attr, u, batch, W1, b1, W2, b2):
    del edge_index, edge_attr
    psums = _sc_segment_sums(x, batch)
    ids2d = jnp.pad(batch.reshape(IDS_R, IDS_C), ((0, 0), (0, 128 - IDS_C)),
                    constant_values=G)
    inv_cnt = pl.pallas_call(
        _count_body,
        out_shape=jax.ShapeDtypeStruct((G, 1), jnp.float32),
    )(ids2d)
    out = pl.pallas_call(
        _mlp_body,
        out_shape=jax.ShapeDtypeStruct((G, W2.shape[1]), jnp.float32),
    )(psums, inv_cnt, u, W1[:DU], W1[DU:], b1.reshape(1, -1), W2,
      b2.reshape(1, -1))
    return out
